# initial kernel scaffold (unmeasured)
import jax
import jax.numpy as jnp
from jax import lax
from jax.experimental import pallas as pl
from jax.experimental.pallas import tpu as pltpu

N_DEV = 4
M_OUT = 512
D = 2048
F = 8192
FB = 512
NF = F // FB


def kernel(x, dy):
    def body(x_ref, dy_hbm, out_ref, comm_ref, dyv_ref, send_sems, recv_sems,
             dma_sem):
        my = lax.axis_index("i")
        left = lax.rem(my + (N_DEV - 1), N_DEV)
        right = lax.rem(my + 1, N_DEV)

        barrier_sem = pltpu.get_barrier_semaphore()
        for nbr in (left, right):
            pl.semaphore_signal(
                barrier_sem, inc=1,
                device_id=(nbr,), device_id_type=pl.DeviceIdType.MESH,
            )
        pl.semaphore_wait(barrier_sem, 2)

        def add_partial(chunk_j, slot, init):
            xj = x_ref[:, pl.ds(chunk_j * M_OUT, M_OUT)].astype(jnp.bfloat16)
            for f in range(NF):
                cp = pltpu.make_async_copy(
                    dy_hbm.at[:, pl.ds(f * FB, FB)], dyv_ref, dma_sem)
                cp.start()
                cp.wait()
                mm = lax.dot_general(
                    xj, dyv_ref[...].astype(jnp.bfloat16),
                    dimension_numbers=(((0,), (0,)), ((), ())),
                    preferred_element_type=jnp.float32,
                )
                if init:
                    comm_ref[slot, :, pl.ds(f * FB, FB)] = mm
                else:
                    comm_ref[slot, :, pl.ds(f * FB, FB)] = (
                        comm_ref[slot, :, pl.ds(f * FB, FB)] + mm)

        add_partial(lax.rem(my + 3, N_DEV), 0, init=True)
        for h in range(N_DEV - 1):
            send_slot = h % 2
            recv_slot = (h + 1) % 2
            rdma = pltpu.make_async_remote_copy(
                src_ref=comm_ref.at[send_slot],
                dst_ref=comm_ref.at[recv_slot],
                send_sem=send_sems.at[send_slot],
                recv_sem=recv_sems.at[recv_slot],
                device_id=(right,),
                device_id_type=pl.DeviceIdType.MESH,
            )
            rdma.start()
            rdma.wait()
            recv_chunk = lax.rem(my + (N_DEV - 2 - h), N_DEV)
            add_partial(recv_chunk, recv_slot, init=False)

        out_cp = pltpu.make_async_copy(comm_ref.at[1], out_ref, dma_sem)
        out_cp.start()
        out_cp.wait()

    return pl.pallas_call(
        body,
        out_shape=jax.ShapeDtypeStruct((M_OUT, F), jnp.float32),
        in_specs=[
            pl.BlockSpec(memory_space=pltpu.VMEM),
            pl.BlockSpec(memory_space=pltpu.ANY),
        ],
        out_specs=pl.BlockSpec(memory_space=pltpu.ANY),
        scratch_shapes=[
            pltpu.VMEM((2, M_OUT, F), jnp.float32),
            pltpu.VMEM((D, FB), jnp.float32),
            pltpu.SemaphoreType.DMA((2,)),
            pltpu.SemaphoreType.DMA((2,)),
            pltpu.SemaphoreType.DMA,
        ],
        compiler_params=pltpu.CompilerParams(collective_id=0),
    )(x, dy)


# baseline (device time: 809597 ns/iter reference)
import jax
import jax.numpy as jnp
from jax import lax
from jax.experimental import pallas as pl
from jax.experimental.pallas import tpu as pltpu

N_DEV = 4
M_OUT = 512
D = 2048
F = 8192
FB = 512
NF = F // FB


def kernel(x, dy):
    def body(x_hbm, dy_hbm, out_ref, comm_ref, xv_ref, dyv_ref, send_sems,
             recv_sems, dma_sem):
        my = lax.axis_index("i")
        left = lax.rem(my + (N_DEV - 1), N_DEV)
        right = lax.rem(my + 1, N_DEV)

        barrier_sem = pltpu.get_barrier_semaphore()
        for nbr in (left, right):
            pl.semaphore_signal(
                barrier_sem, inc=1,
                device_id=(nbr,), device_id_type=pl.DeviceIdType.MESH,
            )
        pl.semaphore_wait(barrier_sem, 2)

        def add_partial(chunk_j, slot, init):
            cpx = pltpu.make_async_copy(
                x_hbm.at[:, pl.ds(chunk_j * M_OUT, M_OUT)], xv_ref, dma_sem)
            cpx.start()
            cpx.wait()
            xj = xv_ref[...].astype(jnp.bfloat16)

            def fbody(f, carry):
                cp = pltpu.make_async_copy(
                    dy_hbm.at[:, pl.ds(f * FB, FB)], dyv_ref, dma_sem)
                cp.start()
                cp.wait()
                mm = lax.dot_general(
                    xj, dyv_ref[...].astype(jnp.bfloat16),
                    dimension_numbers=(((0,), (0,)), ((), ())),
                    preferred_element_type=jnp.float32,
                )
                if init:
                    comm_ref[slot, :, pl.ds(f * FB, FB)] = mm
                else:
                    comm_ref[slot, :, pl.ds(f * FB, FB)] = (
                        comm_ref[slot, :, pl.ds(f * FB, FB)] + mm)
                return carry

            lax.fori_loop(0, NF, fbody, 0)

        add_partial(lax.rem(my + 3, N_DEV), 0, init=True)
        for h in range(N_DEV - 1):
            send_slot = h % 2
            recv_slot = (h + 1) % 2
            rdma = pltpu.make_async_remote_copy(
                src_ref=comm_ref.at[send_slot],
                dst_ref=comm_ref.at[recv_slot],
                send_sem=send_sems.at[send_slot],
                recv_sem=recv_sems.at[recv_slot],
                device_id=(right,),
                device_id_type=pl.DeviceIdType.MESH,
            )
            rdma.start()
            rdma.wait()
            recv_chunk = lax.rem(my + (N_DEV - 2 - h), N_DEV)
            add_partial(recv_chunk, recv_slot, init=False)

        out_cp = pltpu.make_async_copy(comm_ref.at[1], out_ref, dma_sem)
        out_cp.start()
        out_cp.wait()

    return pl.pallas_call(
        body,
        out_shape=jax.ShapeDtypeStruct((M_OUT, F), jnp.float32),
        in_specs=[
            pl.BlockSpec(memory_space=pl.ANY),
            pl.BlockSpec(memory_space=pl.ANY),
        ],
        out_specs=pl.BlockSpec(memory_space=pl.ANY),
        scratch_shapes=[
            pltpu.VMEM((2, M_OUT, F), jnp.float32),
            pltpu.VMEM((D, M_OUT), jnp.float32),
            pltpu.VMEM((D, FB), jnp.float32),
            pltpu.SemaphoreType.DMA((2,)),
            pltpu.SemaphoreType.DMA((2,)),
            pltpu.SemaphoreType.DMA,
        ],
        compiler_params=pltpu.CompilerParams(
            collective_id=0,
            vmem_limit_bytes=56 * 1024 * 1024,
        ),
    )(x, dy)


# device time: 358029 ns/iter; 2.2613x vs baseline; 2.2613x over previous
import jax
import jax.numpy as jnp
from jax import lax
from jax.experimental import pallas as pl
from jax.experimental.pallas import tpu as pltpu

N_DEV = 4
M_OUT = 512
D = 2048
F = 8192
HALF = F // 2
NSUB = 4
SUB = HALF // NSUB
FB = 512
NFB = SUB // FB


def kernel(x, dy):
    def body(x_hbm, dy_hbm, out_ref, comm_cw, comm_ccw, xbf, xv, dyv,
             cw_send, cw_recv, ccw_send, ccw_recv, dma_sem, out_sems):
        my = lax.axis_index("i")
        left = lax.rem(my + (N_DEV - 1), N_DEV)
        right = lax.rem(my + 1, N_DEV)

        barrier_sem = pltpu.get_barrier_semaphore()
        for nbr in (left, right):
            pl.semaphore_signal(
                barrier_sem, inc=1,
                device_id=(nbr,), device_id_type=pl.DeviceIdType.MESH,
            )
        pl.semaphore_wait(barrier_sem, 2)

        for j in range(N_DEV):
            cp = pltpu.make_async_copy(
                x_hbm.at[:, pl.ds(j * M_OUT, M_OUT)], xv, dma_sem)
            cp.start()
            cp.wait()
            xbf[:, pl.ds(j * M_OUT, M_OUT)] = xv[...].astype(jnp.bfloat16)

        def add_sub(comm, slot, chunk_j, s, ccw_dir, init):
            xj = xbf[:, pl.ds(chunk_j * M_OUT, M_OUT)]
            for b in range(NFB):
                rel = s * SUB + b * FB
                col = rel + (HALF if ccw_dir else 0)
                cp = pltpu.make_async_copy(
                    dy_hbm.at[:, pl.ds(col, FB)], dyv, dma_sem)
                cp.start()
                cp.wait()
                mm = lax.dot_general(
                    xj, dyv[...].astype(jnp.bfloat16),
                    dimension_numbers=(((0,), (0,)), ((), ())),
                    preferred_element_type=jnp.float32,
                )
                if init:
                    comm[slot, :, pl.ds(rel, FB)] = mm
                else:
                    comm[slot, :, pl.ds(rel, FB)] = (
                        comm[slot, :, pl.ds(rel, FB)] + mm)

        def mk_rdma(comm, S, R, s, send_sems, recv_sems, dst_dev):
            return pltpu.make_async_remote_copy(
                src_ref=comm.at[S, :, pl.ds(s * SUB, SUB)],
                dst_ref=comm.at[R, :, pl.ds(s * SUB, SUB)],
                send_sem=send_sems.at[S, s],
                recv_sem=recv_sems.at[R, s],
                device_id=(dst_dev,),
                device_id_type=pl.DeviceIdType.MESH,
            )

        rdmas = {}

        cw_c0 = lax.rem(my + 3, N_DEV)
        ccw_c0 = lax.rem(my + 1, N_DEV)
        for s in range(NSUB):
            add_sub(comm_cw, 0, cw_c0, s, False, init=True)
            r = mk_rdma(comm_cw, 0, 1, s, cw_send, cw_recv, right)
            r.start()
            rdmas[("cw", 0, s)] = r
            add_sub(comm_ccw, 0, ccw_c0, s, True, init=True)
            r = mk_rdma(comm_ccw, 0, 1, s, ccw_send, ccw_recv, left)
            r.start()
            rdmas[("ccw", 0, s)] = r

        for h in range(N_DEV - 1):
            S, R = h % 2, (h + 1) % 2
            if h > 0:
                for s in range(NSUB):
                    r = mk_rdma(comm_cw, S, R, s, cw_send, cw_recv, right)
                    r.start()
                    rdmas[("cw", h, s)] = r
                    r = mk_rdma(comm_ccw, S, R, s, ccw_send, ccw_recv, left)
                    r.start()
                    rdmas[("ccw", h, s)] = r
            cw_c = lax.rem(my + (N_DEV - 2 - h), N_DEV)
            ccw_c = lax.rem(my + 2 + h, N_DEV)
            for s in range(NSUB):
                rdmas[("cw", h, s)].wait_recv()
                add_sub(comm_cw, R, cw_c, s, False, init=False)
                rdmas[("ccw", h, s)].wait_recv()
                add_sub(comm_ccw, R, ccw_c, s, True, init=False)
            for s in range(NSUB):
                rdmas[("cw", h, s)].wait_send()
                rdmas[("ccw", h, s)].wait_send()

        cp0 = pltpu.make_async_copy(
            comm_cw.at[1], out_ref.at[:, pl.ds(0, HALF)], out_sems.at[0])
        cp1 = pltpu.make_async_copy(
            comm_ccw.at[1], out_ref.at[:, pl.ds(HALF, HALF)], out_sems.at[1])
        cp0.start()
        cp1.start()
        cp0.wait()
        cp1.wait()

    return pl.pallas_call(
        body,
        out_shape=jax.ShapeDtypeStruct((M_OUT, F), jnp.float32),
        in_specs=[
            pl.BlockSpec(memory_space=pl.ANY),
            pl.BlockSpec(memory_space=pl.ANY),
        ],
        out_specs=pl.BlockSpec(memory_space=pl.ANY),
        scratch_shapes=[
            pltpu.VMEM((2, M_OUT, HALF), jnp.float32),
            pltpu.VMEM((2, M_OUT, HALF), jnp.float32),
            pltpu.VMEM((D, D), jnp.bfloat16),
            pltpu.VMEM((D, M_OUT), jnp.float32),
            pltpu.VMEM((D, FB), jnp.float32),
            pltpu.SemaphoreType.DMA((2, NSUB)),
            pltpu.SemaphoreType.DMA((2, NSUB)),
            pltpu.SemaphoreType.DMA((2, NSUB)),
            pltpu.SemaphoreType.DMA((2, NSUB)),
            pltpu.SemaphoreType.DMA,
            pltpu.SemaphoreType.DMA((2,)),
        ],
        compiler_params=pltpu.CompilerParams(
            collective_id=0,
            vmem_limit_bytes=60 * 1024 * 1024,
        ),
    )(x, dy)


# device time: 320876 ns/iter; 2.5231x vs baseline; 1.1158x over previous
import jax
import jax.numpy as jnp
from jax import lax
from jax.experimental import pallas as pl
from jax.experimental.pallas import tpu as pltpu

N_DEV = 4
M_OUT = 512
D = 2048
F = 8192
HALF = F // 2
NSUB = 4
SUB = HALF // NSUB
FB = 512
NFB = SUB // FB


def kernel(x, dy):
    def body(x_hbm, dy_hbm, out_ref, comm_cw, comm_ccw, xbf, xv, dyv,
             cw_send, cw_recv, ccw_send, ccw_recv, cw_credit, ccw_credit,
             dma_sem, out_sems):
        my = lax.axis_index("i")
        left = lax.rem(my + (N_DEV - 1), N_DEV)
        right = lax.rem(my + 1, N_DEV)

        barrier_sem = pltpu.get_barrier_semaphore()
        for nbr in (left, right):
            pl.semaphore_signal(
                barrier_sem, inc=1,
                device_id=(nbr,), device_id_type=pl.DeviceIdType.MESH,
            )
        pl.semaphore_wait(barrier_sem, 2)

        for j in range(N_DEV):
            cp = pltpu.make_async_copy(
                x_hbm.at[:, pl.ds(j * M_OUT, M_OUT)], xv, dma_sem)
            cp.start()
            cp.wait()
            xbf[:, pl.ds(j * M_OUT, M_OUT)] = xv[...].astype(jnp.bfloat16)

        def add_sub(comm, slot, chunk_j, s, ccw_dir, init):
            xj = xbf[:, pl.ds(chunk_j * M_OUT, M_OUT)]
            for b in range(NFB):
                rel = s * SUB + b * FB
                col = rel + (HALF if ccw_dir else 0)
                cp = pltpu.make_async_copy(
                    dy_hbm.at[:, pl.ds(col, FB)], dyv, dma_sem)
                cp.start()
                cp.wait()
                mm = lax.dot_general(
                    xj, dyv[...].astype(jnp.bfloat16),
                    dimension_numbers=(((0,), (0,)), ((), ())),
                    preferred_element_type=jnp.float32,
                )
                if init:
                    comm[slot, :, pl.ds(rel, FB)] = mm
                else:
                    comm[slot, :, pl.ds(rel, FB)] = (
                        comm[slot, :, pl.ds(rel, FB)] + mm)

        def mk_rdma(comm, src_slot, dst_slot, s, send_sems, recv_sems,
                    dst_dev):
            return pltpu.make_async_remote_copy(
                src_ref=comm.at[src_slot, :, pl.ds(s * SUB, SUB)],
                dst_ref=comm.at[dst_slot, :, pl.ds(s * SUB, SUB)],
                send_sem=send_sems.at[src_slot, s],
                recv_sem=recv_sems.at[dst_slot, s],
                device_id=(dst_dev,),
                device_id_type=pl.DeviceIdType.MESH,
            )

        rdmas = {}
        out_cps = []

        cw_c0 = lax.rem(my + 3, N_DEV)
        ccw_c0 = lax.rem(my + 1, N_DEV)
        for s in range(NSUB):
            add_sub(comm_cw, 0, cw_c0, s, False, init=True)
            r = mk_rdma(comm_cw, 0, 1, s, cw_send, cw_recv, right)
            r.start()
            rdmas[("cw", 0, s)] = r
            add_sub(comm_ccw, 0, ccw_c0, s, True, init=True)
            r = mk_rdma(comm_ccw, 0, 1, s, ccw_send, ccw_recv, left)
            r.start()
            rdmas[("ccw", 0, s)] = r

        dirs = (
            ("cw", comm_cw, cw_send, cw_recv, cw_credit, right, left),
            ("ccw", comm_ccw, ccw_send, ccw_recv, ccw_credit, left, right),
        )
        for h in range(N_DEV - 1):
            S, R = h % 2, (h + 1) % 2
            cw_c = lax.rem(my + (N_DEV - 2 - h), N_DEV)
            ccw_c = lax.rem(my + 2 + h, N_DEV)
            chunks = {"cw": cw_c, "ccw": ccw_c}
            for s in range(NSUB):
                for (name, comm, send_sems, recv_sems, credit, dst_dev,
                     upstream) in dirs:
                    r = rdmas[(name, h, s)]
                    r.wait_recv()
                    if h < N_DEV - 2:
                        r.wait_send()
                        pl.semaphore_signal(
                            credit.at[s], inc=1,
                            device_id=(upstream,),
                            device_id_type=pl.DeviceIdType.MESH,
                        )
                    add_sub(comm, R, chunks[name], s, name == "ccw",
                            init=False)
                    if h < N_DEV - 2:
                        pl.semaphore_wait(credit.at[s], 1)
                        r2 = mk_rdma(comm, R, S, s, send_sems, recv_sems,
                                     dst_dev)
                        r2.start()
                        rdmas[(name, h + 1, s)] = r2
                    else:
                        r.wait_send()
                        d = 0 if name == "cw" else 1
                        off = 0 if name == "cw" else HALF
                        cp = pltpu.make_async_copy(
                            comm.at[1, :, pl.ds(s * SUB, SUB)],
                            out_ref.at[:, pl.ds(off + s * SUB, SUB)],
                            out_sems.at[d, s])
                        cp.start()
                        out_cps.append(cp)

        for cp in out_cps:
            cp.wait()

    return pl.pallas_call(
        body,
        out_shape=jax.ShapeDtypeStruct((M_OUT, F), jnp.float32),
        in_specs=[
            pl.BlockSpec(memory_space=pl.ANY),
            pl.BlockSpec(memory_space=pl.ANY),
        ],
        out_specs=pl.BlockSpec(memory_space=pl.ANY),
        scratch_shapes=[
            pltpu.VMEM((2, M_OUT, HALF), jnp.float32),
            pltpu.VMEM((2, M_OUT, HALF), jnp.float32),
            pltpu.VMEM((D, D), jnp.bfloat16),
            pltpu.VMEM((D, M_OUT), jnp.float32),
            pltpu.VMEM((D, FB), jnp.float32),
            pltpu.SemaphoreType.DMA((2, NSUB)),
            pltpu.SemaphoreType.DMA((2, NSUB)),
            pltpu.SemaphoreType.DMA((2, NSUB)),
            pltpu.SemaphoreType.DMA((2, NSUB)),
            pltpu.SemaphoreType.REGULAR((NSUB,)),
            pltpu.SemaphoreType.REGULAR((NSUB,)),
            pltpu.SemaphoreType.DMA,
            pltpu.SemaphoreType.DMA((2, NSUB)),
        ],
        compiler_params=pltpu.CompilerParams(
            collective_id=0,
            vmem_limit_bytes=60 * 1024 * 1024,
        ),
    )(x, dy)


# device time: 313965 ns/iter; 2.5786x vs baseline; 1.0220x over previous
import jax
import jax.numpy as jnp
from jax import lax
from jax.experimental import pallas as pl
from jax.experimental.pallas import tpu as pltpu

N_DEV = 4
M_OUT = 512
D = 2048
F = 8192
HALF = F // 2
NSUB = 4
SUB = HALF // NSUB
FB = 512
NFB = SUB // FB


def kernel(x, dy):
    def body(x_hbm, dy_hbm, out_ref, comm_cw, comm_ccw, xbf, xv, dyv,
             cw_send, cw_recv, ccw_send, ccw_recv, cw_credit, ccw_credit,
             dma_sems, out_sems):
        my = lax.axis_index("i")
        left = lax.rem(my + (N_DEV - 1), N_DEV)
        right = lax.rem(my + 1, N_DEV)

        barrier_sem = pltpu.get_barrier_semaphore()
        for nbr in (left, right):
            pl.semaphore_signal(
                barrier_sem, inc=1,
                device_id=(nbr,), device_id_type=pl.DeviceIdType.MESH,
            )
        pl.semaphore_wait(barrier_sem, 2)

        for j in range(N_DEV):
            cp = pltpu.make_async_copy(
                x_hbm.at[:, pl.ds(j * M_OUT, M_OUT)], xv, dma_sems.at[0])
            cp.start()
            cp.wait()
            xbf[:, pl.ds(j * M_OUT, M_OUT)] = xv[...].astype(jnp.bfloat16)

        def add_sub(comm, slot, chunk_j, s, ccw_dir, init):
            xj = xbf[:, pl.ds(chunk_j * M_OUT, M_OUT)]
            cps = []
            for b in range(NFB):
                col = s * SUB + b * FB + (HALF if ccw_dir else 0)
                cp = pltpu.make_async_copy(
                    dy_hbm.at[:, pl.ds(col, FB)], dyv.at[b % 2],
                    dma_sems.at[b % 2])
                cp.start()
                cps.append(cp)
            for b in range(NFB):
                rel = s * SUB + b * FB
                cps[b].wait()
                mm = lax.dot_general(
                    xj, dyv[b % 2].astype(jnp.bfloat16),
                    dimension_numbers=(((0,), (0,)), ((), ())),
                    preferred_element_type=jnp.float32,
                )
                if init:
                    comm[slot, :, pl.ds(rel, FB)] = mm
                else:
                    comm[slot, :, pl.ds(rel, FB)] = (
                        comm[slot, :, pl.ds(rel, FB)] + mm)

        def mk_rdma(comm, src_slot, dst_slot, s, send_sems, recv_sems,
                    dst_dev):
            return pltpu.make_async_remote_copy(
                src_ref=comm.at[src_slot, :, pl.ds(s * SUB, SUB)],
                dst_ref=comm.at[dst_slot, :, pl.ds(s * SUB, SUB)],
                send_sem=send_sems.at[src_slot, s],
                recv_sem=recv_sems.at[dst_slot, s],
                device_id=(dst_dev,),
                device_id_type=pl.DeviceIdType.MESH,
            )

        rdmas = {}
        out_cps = []

        cw_c0 = lax.rem(my + 3, N_DEV)
        ccw_c0 = lax.rem(my + 1, N_DEV)
        for s in range(NSUB):
            add_sub(comm_cw, 0, cw_c0, s, False, init=True)
            r = mk_rdma(comm_cw, 0, 1, s, cw_send, cw_recv, right)
            r.start()
            rdmas[("cw", 0, s)] = r
            add_sub(comm_ccw, 0, ccw_c0, s, True, init=True)
            r = mk_rdma(comm_ccw, 0, 1, s, ccw_send, ccw_recv, left)
            r.start()
            rdmas[("ccw", 0, s)] = r

        dirs = (
            ("cw", comm_cw, cw_send, cw_recv, cw_credit, right, left),
            ("ccw", comm_ccw, ccw_send, ccw_recv, ccw_credit, left, right),
        )
        for h in range(N_DEV - 1):
            S, R = h % 2, (h + 1) % 2
            cw_c = lax.rem(my + (N_DEV - 2 - h), N_DEV)
            ccw_c = lax.rem(my + 2 + h, N_DEV)
            chunks = {"cw": cw_c, "ccw": ccw_c}
            for s in range(NSUB):
                for (name, comm, send_sems, recv_sems, credit, dst_dev,
                     upstream) in dirs:
                    r = rdmas[(name, h, s)]
                    r.wait_recv()
                    if h < N_DEV - 2:
                        r.wait_send()
                        pl.semaphore_signal(
                            credit.at[s], inc=1,
                            device_id=(upstream,),
                            device_id_type=pl.DeviceIdType.MESH,
                        )
                    add_sub(comm, R, chunks[name], s, name == "ccw",
                            init=False)
                    if h < N_DEV - 2:
                        pl.semaphore_wait(credit.at[s], 1)
                        r2 = mk_rdma(comm, R, S, s, send_sems, recv_sems,
                                     dst_dev)
                        r2.start()
                        rdmas[(name, h + 1, s)] = r2
                    else:
                        r.wait_send()
                        d = 0 if name == "cw" else 1
                        off = 0 if name == "cw" else HALF
                        cp = pltpu.make_async_copy(
                            comm.at[1, :, pl.ds(s * SUB, SUB)],
                            out_ref.at[:, pl.ds(off + s * SUB, SUB)],
                            out_sems.at[d, s])
                        cp.start()
                        out_cps.append(cp)

        for cp in out_cps:
            cp.wait()

    return pl.pallas_call(
        body,
        out_shape=jax.ShapeDtypeStruct((M_OUT, F), jnp.float32),
        in_specs=[
            pl.BlockSpec(memory_space=pl.ANY),
            pl.BlockSpec(memory_space=pl.ANY),
        ],
        out_specs=pl.BlockSpec(memory_space=pl.ANY),
        scratch_shapes=[
            pltpu.VMEM((2, M_OUT, HALF), jnp.float32),
            pltpu.VMEM((2, M_OUT, HALF), jnp.float32),
            pltpu.VMEM((D, D), jnp.bfloat16),
            pltpu.VMEM((D, M_OUT), jnp.float32),
            pltpu.VMEM((2, D, FB), jnp.float32),
            pltpu.SemaphoreType.DMA((2, NSUB)),
            pltpu.SemaphoreType.DMA((2, NSUB)),
            pltpu.SemaphoreType.DMA((2, NSUB)),
            pltpu.SemaphoreType.DMA((2, NSUB)),
            pltpu.SemaphoreType.REGULAR((NSUB,)),
            pltpu.SemaphoreType.REGULAR((NSUB,)),
            pltpu.SemaphoreType.DMA((2,)),
            pltpu.SemaphoreType.DMA((2, NSUB)),
        ],
        compiler_params=pltpu.CompilerParams(
            collective_id=0,
            vmem_limit_bytes=60 * 1024 * 1024,
        ),
    )(x, dy)


# device time: 306760 ns/iter; 2.6392x vs baseline; 1.0235x over previous
import jax
import jax.numpy as jnp
from jax import lax
from jax.experimental import pallas as pl
from jax.experimental.pallas import tpu as pltpu

N_DEV = 4
M_OUT = 512
D = 2048
F = 8192
HALF = F // 2
NSUB = 4
SUB = HALF // NSUB
FB = 512
NFB = SUB // FB


def kernel(x, dy):
    def body(x_hbm, dy_hbm, out_ref, comm_cw, comm_ccw, xbf, dyv, acc,
             cw_send, cw_recv, ccw_send, ccw_recv, cw_credit, ccw_credit,
             dma_sems, out_sems):
        my = lax.axis_index("i")
        left = lax.rem(my + (N_DEV - 1), N_DEV)
        right = lax.rem(my + 1, N_DEV)

        barrier_sem = pltpu.get_barrier_semaphore()
        for nbr in (left, right):
            pl.semaphore_signal(
                barrier_sem, inc=1,
                device_id=(nbr,), device_id_type=pl.DeviceIdType.MESH,
            )
        pl.semaphore_wait(barrier_sem, 2)

        xcps = {}
        for j in range(2):
            xcps[j] = pltpu.make_async_copy(
                x_hbm.at[:, pl.ds(j * M_OUT, M_OUT)], dyv.at[j % 2],
                dma_sems.at[j % 2])
            xcps[j].start()
        for j in range(N_DEV):
            xcps[j].wait()
            xbf[:, pl.ds(j * M_OUT, M_OUT)] = (
                dyv[j % 2].astype(jnp.bfloat16))
            if j + 2 < N_DEV:
                xcps[j + 2] = pltpu.make_async_copy(
                    x_hbm.at[:, pl.ds((j + 2) * M_OUT, M_OUT)],
                    dyv.at[j % 2], dma_sems.at[j % 2])
                xcps[j + 2].start()

        def stream_mm(chunk_j, s, ccw_dir, emit):
            xj = xbf[:, pl.ds(chunk_j * M_OUT, M_OUT)]
            cps = []
            for b in range(NFB):
                col = s * SUB + b * FB + (HALF if ccw_dir else 0)
                cp = pltpu.make_async_copy(
                    dy_hbm.at[:, pl.ds(col, FB)], dyv.at[b % 2],
                    dma_sems.at[b % 2])
                cp.start()
                cps.append(cp)
            for b in range(NFB):
                cps[b].wait()
                mm = lax.dot_general(
                    xj, dyv[b % 2].astype(jnp.bfloat16),
                    dimension_numbers=(((0,), (0,)), ((), ())),
                    preferred_element_type=jnp.float32,
                )
                emit(b, mm)

        def init_sub(comm, chunk_j, s, ccw_dir):
            def emit(b, mm):
                comm[0, :, pl.ds(s * SUB + b * FB, FB)] = mm
            stream_mm(chunk_j, s, ccw_dir, emit)

        def acc_sub(d, chunk_j, s, ccw_dir):
            def emit(b, mm):
                acc[d, :, pl.ds(b * FB, FB)] = mm
            stream_mm(chunk_j, s, ccw_dir, emit)

        def mk_rdma(comm, src_slot, dst_slot, s, send_sems, recv_sems,
                    dst_dev):
            return pltpu.make_async_remote_copy(
                src_ref=comm.at[src_slot, :, pl.ds(s * SUB, SUB)],
                dst_ref=comm.at[dst_slot, :, pl.ds(s * SUB, SUB)],
                send_sem=send_sems.at[src_slot, s],
                recv_sem=recv_sems.at[dst_slot, s],
                device_id=(dst_dev,),
                device_id_type=pl.DeviceIdType.MESH,
            )

        rdmas = {}
        out_cps = []

        cw_c0 = lax.rem(my + 3, N_DEV)
        ccw_c0 = lax.rem(my + 1, N_DEV)
        for s in range(NSUB):
            init_sub(comm_cw, cw_c0, s, False)
            r = mk_rdma(comm_cw, 0, 1, s, cw_send, cw_recv, right)
            r.start()
            rdmas[("cw", 0, s)] = r
            init_sub(comm_ccw, ccw_c0, s, True)
            r = mk_rdma(comm_ccw, 0, 1, s, ccw_send, ccw_recv, left)
            r.start()
            rdmas[("ccw", 0, s)] = r

        dirs = (
            ("cw", comm_cw, cw_send, cw_recv, cw_credit, right, left, 0),
            ("ccw", comm_ccw, ccw_send, ccw_recv, ccw_credit, left, right,
             1),
        )
        for h in range(N_DEV - 1):
            S, R = h % 2, (h + 1) % 2
            cw_c = lax.rem(my + (N_DEV - 2 - h), N_DEV)
            ccw_c = lax.rem(my + 2 + h, N_DEV)
            chunks = {"cw": cw_c, "ccw": ccw_c}
            for s in range(NSUB):
                for (name, comm, send_sems, recv_sems, credit, dst_dev,
                     upstream, d) in dirs:
                    acc_sub(d, chunks[name], s, name == "ccw")
                    r = rdmas[(name, h, s)]
                    r.wait_recv()
                    if h < N_DEV - 2:
                        r.wait_send()
                        pl.semaphore_signal(
                            credit.at[s], inc=1,
                            device_id=(upstream,),
                            device_id_type=pl.DeviceIdType.MESH,
                        )
                    comm[R, :, pl.ds(s * SUB, SUB)] = (
                        comm[R, :, pl.ds(s * SUB, SUB)] + acc[d])
                    if h < N_DEV - 2:
                        pl.semaphore_wait(credit.at[s], 1)
                        r2 = mk_rdma(comm, R, S, s, send_sems, recv_sems,
                                     dst_dev)
                        r2.start()
                        rdmas[(name, h + 1, s)] = r2
                    else:
                        r.wait_send()
                        off = 0 if name == "cw" else HALF
                        cp = pltpu.make_async_copy(
                            comm.at[1, :, pl.ds(s * SUB, SUB)],
                            out_ref.at[:, pl.ds(off + s * SUB, SUB)],
                            out_sems.at[d, s])
                        cp.start()
                        out_cps.append(cp)

        for cp in out_cps:
            cp.wait()

    return pl.pallas_call(
        body,
        out_shape=jax.ShapeDtypeStruct((M_OUT, F), jnp.float32),
        in_specs=[
            pl.BlockSpec(memory_space=pl.ANY),
            pl.BlockSpec(memory_space=pl.ANY),
        ],
        out_specs=pl.BlockSpec(memory_space=pl.ANY),
        scratch_shapes=[
            pltpu.VMEM((2, M_OUT, HALF), jnp.float32),
            pltpu.VMEM((2, M_OUT, HALF), jnp.float32),
            pltpu.VMEM((D, D), jnp.bfloat16),
            pltpu.VMEM((2, D, FB), jnp.float32),
            pltpu.VMEM((2, M_OUT, SUB), jnp.float32),
            pltpu.SemaphoreType.DMA((2, NSUB)),
            pltpu.SemaphoreType.DMA((2, NSUB)),
            pltpu.SemaphoreType.DMA((2, NSUB)),
            pltpu.SemaphoreType.DMA((2, NSUB)),
            pltpu.SemaphoreType.REGULAR((NSUB,)),
            pltpu.SemaphoreType.REGULAR((NSUB,)),
            pltpu.SemaphoreType.DMA((2,)),
            pltpu.SemaphoreType.DMA((2, NSUB)),
        ],
        compiler_params=pltpu.CompilerParams(
            collective_id=0,
            vmem_limit_bytes=60 * 1024 * 1024,
        ),
    )(x, dy)


# device time: 210545 ns/iter; 3.8452x vs baseline; 1.4570x over previous
import jax
import jax.numpy as jnp
from jax import lax
from jax.experimental import pallas as pl
from jax.experimental.pallas import tpu as pltpu

N_DEV = 4
M_OUT = 512
D = 2048
F = 8192
HALF = F // 2
NSUB = 4
SUB = HALF // NSUB
FB = 512
NFB = SUB // FB


def kernel(x, dy):
    def body(x_hbm, dy_hbm, out_ref, comm_cw, comm_ccw, xbf, dyv, acc,
             out_stage, cw_send, cw_recv, ccw_send, ccw_recv, cw_credit,
             ccw_credit, dma_sems, out_sems):
        my = lax.axis_index("i")
        left = lax.rem(my + (N_DEV - 1), N_DEV)
        right = lax.rem(my + 1, N_DEV)

        barrier_sem = pltpu.get_barrier_semaphore()
        for nbr in (left, right):
            pl.semaphore_signal(
                barrier_sem, inc=1,
                device_id=(nbr,), device_id_type=pl.DeviceIdType.MESH,
            )
        pl.semaphore_wait(barrier_sem, 2)

        xcps = {}
        for j in range(2):
            xcps[j] = pltpu.make_async_copy(
                x_hbm.at[:, pl.ds(j * M_OUT, M_OUT)], dyv.at[j % 2],
                dma_sems.at[j % 2])
            xcps[j].start()
        for j in range(N_DEV):
            xcps[j].wait()
            xbf[:, pl.ds(j * M_OUT, M_OUT)] = (
                dyv[j % 2].astype(jnp.bfloat16))
            if j + 2 < N_DEV:
                xcps[j + 2] = pltpu.make_async_copy(
                    x_hbm.at[:, pl.ds((j + 2) * M_OUT, M_OUT)],
                    dyv.at[j % 2], dma_sems.at[j % 2])
                xcps[j + 2].start()

        def stream_mm(chunk_j, s, ccw_dir, emit):
            xj = xbf[:, pl.ds(chunk_j * M_OUT, M_OUT)]
            cps = []
            for b in range(NFB):
                col = s * SUB + b * FB + (HALF if ccw_dir else 0)
                cp = pltpu.make_async_copy(
                    dy_hbm.at[:, pl.ds(col, FB)], dyv.at[b % 2],
                    dma_sems.at[b % 2])
                cp.start()
                cps.append(cp)
            for b in range(NFB):
                cps[b].wait()
                mm = lax.dot_general(
                    xj, dyv[b % 2].astype(jnp.bfloat16),
                    dimension_numbers=(((0,), (0,)), ((), ())),
                    preferred_element_type=jnp.float32,
                )
                emit(b, mm)

        def init_sub(comm, chunk_j, s, ccw_dir):
            def emit(b, mm):
                comm[0, :, pl.ds(s * SUB + b * FB, FB)] = (
                    mm.astype(jnp.bfloat16))
            stream_mm(chunk_j, s, ccw_dir, emit)

        def acc_sub(d, chunk_j, s, ccw_dir):
            def emit(b, mm):
                acc[d, :, pl.ds(b * FB, FB)] = mm
            stream_mm(chunk_j, s, ccw_dir, emit)

        def mk_rdma(comm, src_slot, dst_slot, s, send_sems, recv_sems,
                    dst_dev):
            return pltpu.make_async_remote_copy(
                src_ref=comm.at[src_slot, :, pl.ds(s * SUB, SUB)],
                dst_ref=comm.at[dst_slot, :, pl.ds(s * SUB, SUB)],
                send_sem=send_sems.at[src_slot, s],
                recv_sem=recv_sems.at[dst_slot, s],
                device_id=(dst_dev,),
                device_id_type=pl.DeviceIdType.MESH,
            )

        rdmas = {}
        out_cps = []

        cw_c0 = lax.rem(my + 3, N_DEV)
        ccw_c0 = lax.rem(my + 1, N_DEV)
        for s in range(NSUB):
            init_sub(comm_cw, cw_c0, s, False)
            r = mk_rdma(comm_cw, 0, 1, s, cw_send, cw_recv, right)
            r.start()
            rdmas[("cw", 0, s)] = r
            init_sub(comm_ccw, ccw_c0, s, True)
            r = mk_rdma(comm_ccw, 0, 1, s, ccw_send, ccw_recv, left)
            r.start()
            rdmas[("ccw", 0, s)] = r

        dirs = (
            ("cw", comm_cw, cw_send, cw_recv, cw_credit, right, left, 0),
            ("ccw", comm_ccw, ccw_send, ccw_recv, ccw_credit, left, right,
             1),
        )
        for h in range(N_DEV - 1):
            S, R = h % 2, (h + 1) % 2
            cw_c = lax.rem(my + (N_DEV - 2 - h), N_DEV)
            ccw_c = lax.rem(my + 2 + h, N_DEV)
            chunks = {"cw": cw_c, "ccw": ccw_c}
            for s in range(NSUB):
                for (name, comm, send_sems, recv_sems, credit, dst_dev,
                     upstream, d) in dirs:
                    acc_sub(d, chunks[name], s, name == "ccw")
                    r = rdmas[(name, h, s)]
                    r.wait_recv()
                    if h < N_DEV - 2:
                        r.wait_send()
                        pl.semaphore_signal(
                            credit.at[s], inc=1,
                            device_id=(upstream,),
                            device_id_type=pl.DeviceIdType.MESH,
                        )
                    if h < N_DEV - 2:
                        comm[R, :, pl.ds(s * SUB, SUB)] = (
                            comm[R, :, pl.ds(s * SUB, SUB)]
                            + acc[d].astype(jnp.bfloat16))
                        pl.semaphore_wait(credit.at[s], 1)
                        r2 = mk_rdma(comm, R, S, s, send_sems, recv_sems,
                                     dst_dev)
                        r2.start()
                        rdmas[(name, h + 1, s)] = r2
                    else:
                        if s > 0:
                            out_cps[-2].wait()
                        out_stage[d] = (
                            comm[1, :, pl.ds(s * SUB, SUB)].astype(
                                jnp.float32) + acc[d])
                        r.wait_send()
                        off = 0 if name == "cw" else HALF
                        cp = pltpu.make_async_copy(
                            out_stage.at[d],
                            out_ref.at[:, pl.ds(off + s * SUB, SUB)],
                            out_sems.at[d, s])
                        cp.start()
                        out_cps.append(cp)

        for cp in out_cps[-2:]:
            cp.wait()

    return pl.pallas_call(
        body,
        out_shape=jax.ShapeDtypeStruct((M_OUT, F), jnp.float32),
        in_specs=[
            pl.BlockSpec(memory_space=pl.ANY),
            pl.BlockSpec(memory_space=pl.ANY),
        ],
        out_specs=pl.BlockSpec(memory_space=pl.ANY),
        scratch_shapes=[
            pltpu.VMEM((2, M_OUT, HALF), jnp.bfloat16),
            pltpu.VMEM((2, M_OUT, HALF), jnp.bfloat16),
            pltpu.VMEM((D, D), jnp.bfloat16),
            pltpu.VMEM((2, D, FB), jnp.float32),
            pltpu.VMEM((2, M_OUT, SUB), jnp.float32),
            pltpu.VMEM((2, M_OUT, SUB), jnp.float32),
            pltpu.SemaphoreType.DMA((2, NSUB)),
            pltpu.SemaphoreType.DMA((2, NSUB)),
            pltpu.SemaphoreType.DMA((2, NSUB)),
            pltpu.SemaphoreType.DMA((2, NSUB)),
            pltpu.SemaphoreType.REGULAR((NSUB,)),
            pltpu.SemaphoreType.REGULAR((NSUB,)),
            pltpu.SemaphoreType.DMA((2,)),
            pltpu.SemaphoreType.DMA((2, NSUB)),
        ],
        compiler_params=pltpu.CompilerParams(
            collective_id=0,
            vmem_limit_bytes=60 * 1024 * 1024,
        ),
    )(x, dy)
